# baseline (speedup 1.0000x reference)
import functools

import jax
import jax.numpy as jnp
from jax.experimental import pallas as pl
from jax.experimental.pallas import tpu as pltpu


def _fused(n_steps, n_sub, k_count, keys_ref, keys_t_ref, qt_ref, predt_ref,
           out_ref, mag_scr, elem_scr):
    kt = keys_t_ref[...]
    keys = keys_ref[...]
    c = kt.shape[1]
    b2 = jnp.sum(kt * kt, axis=0, keepdims=True)
    k4 = jnp.concatenate([kt, b2], axis=0)
    k5 = jnp.concatenate(
        [keys, jnp.ones((c, 1), jnp.float32)], axis=1
    )
    pid = pl.program_id(0)
    qb = qt_ref.shape[1]
    sb = qb // n_sub
    lanes = mag_scr.shape[1]
    for h in range(n_sub):
        cols = pl.ds(h * sb, sb)
        qt = qt_ref[:, cols]
        q4 = jnp.concatenate(
            [-2.0 * qt, jnp.ones((1, sb), jnp.float32)], axis=0
        )
        s = jax.lax.dot_general(
            k4, q4, (((0,), (0,)), ((), ())),
            preferred_element_type=jnp.float32,
        )
        m = jnp.min(s, axis=0, keepdims=True)
        onehot = (s == m).astype(jnp.float32)
        t4 = jax.lax.dot_general(
            k5, onehot, (((0,), (0,)), ((), ())),
            preferred_element_type=jnp.float32,
        )
        tgt = t4[0:3, :] / t4[3:4, :]
        toff = tgt - qt
        mag = jnp.sqrt(jnp.sum(toff * toff, axis=0, keepdims=True))
        x = predt_ref[:, cols] - toff
        ax = jnp.abs(x)
        e = jnp.where(ax < 1.0, 0.5 * x * x, ax - 0.5)
        elem = jnp.sum(e, axis=0, keepdims=True)
        r0 = (pid * qb + h * sb) // lanes
        mag_scr[pl.ds(r0, sb // lanes), :] = mag.reshape(sb // lanes, lanes)
        elem_scr[pl.ds(r0, sb // lanes), :] = elem.reshape(sb // lanes, lanes)

    @pl.when(pid == n_steps - 1)
    def _loss():
        mag = mag_scr[...]
        bits = jax.lax.bitcast_convert_type(mag, jnp.int32)

        def body(_, lohi):
            lo, hi = lohi
            mid = lo + (hi - lo) // 2
            cnt = jnp.sum((bits <= mid).astype(jnp.int32))
            take = cnt >= k_count
            return jnp.where(take, lo, mid + 1), jnp.where(take, mid, hi)

        _, hi = jax.lax.fori_loop(
            0, 31, body, (jnp.int32(0), jnp.int32(0x7F800000))
        )
        thresh = jax.lax.bitcast_convert_type(hi, jnp.float32)
        mask = (mag <= thresh).astype(jnp.float32)
        denom = jnp.maximum(jnp.sum(mask) * 3.0, 1.0)
        loss = jnp.sum(elem_scr[...] * mask) / denom
        out_ref[...] = jnp.broadcast_to(loss, (1, 1))


@jax.jit
def kernel(pred_off, queries, keys):
    Q, D = queries.shape
    C = keys.shape[0]
    QB = 2048
    N_SUB = 8
    LANES = 256
    keys_t = keys.T
    queries_t = queries.T
    pred_t = pred_off.T

    k_count = int(0.99 * (Q - 1)) + 1

    out = pl.pallas_call(
        functools.partial(_fused, Q // QB, N_SUB, k_count),
        grid=(Q // QB,),
        in_specs=[
            pl.BlockSpec((C, D), lambda i: (0, 0)),
            pl.BlockSpec((D, C), lambda i: (0, 0)),
            pl.BlockSpec((D, QB), lambda i: (0, i)),
            pl.BlockSpec((D, QB), lambda i: (0, i)),
        ],
        out_specs=pl.BlockSpec((1, 1), lambda i: (0, 0)),
        out_shape=jax.ShapeDtypeStruct((1, 1), jnp.float32),
        scratch_shapes=[
            pltpu.VMEM((Q // LANES, LANES), jnp.float32),
            pltpu.VMEM((Q // LANES, LANES), jnp.float32),
        ],
    )(keys, keys_t, queries_t, pred_t)
    return out[0, 0]


# speedup vs baseline: 1.7079x; 1.7079x over previous
import functools

import jax
import jax.numpy as jnp
from jax.experimental import pallas as pl
from jax.experimental.pallas import tpu as pltpu


def _fused(n_steps, n_sub, k_count, keys_ref, keys_t_ref, qt_ref, predt_ref,
           out_ref, mag_scr, elem_scr):
    kt = keys_t_ref[...]
    keys = keys_ref[...]
    c = kt.shape[1]
    b2 = jnp.sum(kt * kt, axis=0, keepdims=True)
    k4 = jnp.concatenate([kt, b2], axis=0)
    k5 = jnp.concatenate(
        [keys, jnp.ones((c, 1), jnp.float32)], axis=1
    )
    pid = pl.program_id(0)
    qb = qt_ref.shape[1]
    sb = qb // n_sub
    lanes = mag_scr.shape[1]
    for h in range(n_sub):
        cols = pl.ds(h * sb, sb)
        qt = qt_ref[:, cols]
        q4 = jnp.concatenate(
            [-2.0 * qt, jnp.ones((1, sb), jnp.float32)], axis=0
        )
        s = jax.lax.dot_general(
            k4, q4, (((0,), (0,)), ((), ())),
            preferred_element_type=jnp.float32,
        )
        m = jnp.min(s, axis=0, keepdims=True)
        onehot = (s == m).astype(jnp.float32)
        t4 = jax.lax.dot_general(
            k5, onehot, (((0,), (0,)), ((), ())),
            preferred_element_type=jnp.float32,
        )
        tgt = t4[0:3, :] / t4[3:4, :]
        toff = tgt - qt
        mag = jnp.sqrt(jnp.sum(toff * toff, axis=0, keepdims=True))
        x = predt_ref[:, cols] - toff
        ax = jnp.abs(x)
        e = jnp.where(ax < 1.0, 0.5 * x * x, ax - 0.5)
        elem = jnp.sum(e, axis=0, keepdims=True)
        r0 = (pid * qb + h * sb) // lanes
        mag_scr[pl.ds(r0, sb // lanes), :] = mag.reshape(sb // lanes, lanes)
        elem_scr[pl.ds(r0, sb // lanes), :] = elem.reshape(sb // lanes, lanes)

    @pl.when(pid == n_steps - 1)
    def _loss():
        mag = mag_scr[...]
        bits = jax.lax.bitcast_convert_type(mag, jnp.int32)

        def body(_, lohi):
            lo, hi = lohi
            mid = lo + (hi - lo) // 2
            cnt = jnp.sum((bits <= mid).astype(jnp.int32))
            take = cnt >= k_count
            return jnp.where(take, lo, mid + 1), jnp.where(take, mid, hi)

        _, hi = jax.lax.fori_loop(
            0, 31, body, (jnp.int32(0), jnp.int32(0x7F800000))
        )
        thresh = jax.lax.bitcast_convert_type(hi, jnp.float32)
        mask = (mag <= thresh).astype(jnp.float32)
        denom = jnp.maximum(jnp.sum(mask) * 3.0, 1.0)
        loss = jnp.sum(elem_scr[...] * mask) / denom
        out_ref[...] = jnp.broadcast_to(loss, (1, 1))


@jax.jit
def kernel(pred_off, queries, keys):
    Q, D = queries.shape
    C = keys.shape[0]
    QB = 2048
    N_SUB = 4
    LANES = 512
    keys_t = keys.T
    queries_t = queries.T
    pred_t = pred_off.T

    k_count = int(0.99 * (Q - 1)) + 1

    out = pl.pallas_call(
        functools.partial(_fused, Q // QB, N_SUB, k_count),
        grid=(Q // QB,),
        in_specs=[
            pl.BlockSpec((C, D), lambda i: (0, 0)),
            pl.BlockSpec((D, C), lambda i: (0, 0)),
            pl.BlockSpec((D, QB), lambda i: (0, i)),
            pl.BlockSpec((D, QB), lambda i: (0, i)),
        ],
        out_specs=pl.BlockSpec((1, 1), lambda i: (0, 0)),
        out_shape=jax.ShapeDtypeStruct((1, 1), jnp.float32),
        scratch_shapes=[
            pltpu.VMEM((Q // LANES, LANES), jnp.float32),
            pltpu.VMEM((Q // LANES, LANES), jnp.float32),
        ],
    )(keys, keys_t, queries_t, pred_t)
    return out[0, 0]


# speedup vs baseline: 1.8513x; 1.0839x over previous
import functools

import jax
import jax.numpy as jnp
from jax.experimental import pallas as pl
from jax.experimental.pallas import tpu as pltpu


def _fused(n_steps, n_sub, k_count, keys_ref, keys_t_ref, qt_ref, predt_ref,
           out_ref, mag_scr, elem_scr):
    kt = keys_t_ref[...]
    keys = keys_ref[...]
    c = kt.shape[1]
    b2 = jnp.sum(kt * kt, axis=0, keepdims=True)
    k4 = jnp.concatenate([kt, b2], axis=0)
    k5 = jnp.concatenate(
        [keys, jnp.ones((c, 1), jnp.float32)], axis=1
    )
    pid = pl.program_id(0)
    qb = qt_ref.shape[1]
    sb = qb // n_sub
    lanes = mag_scr.shape[1]
    for h in range(n_sub):
        cols = pl.ds(h * sb, sb)
        qt = qt_ref[:, cols]
        q4 = jnp.concatenate(
            [-2.0 * qt, jnp.ones((1, sb), jnp.float32)], axis=0
        )
        s = jax.lax.dot_general(
            k4, q4, (((0,), (0,)), ((), ())),
            preferred_element_type=jnp.float32,
        )
        m = jnp.min(s, axis=0, keepdims=True)
        onehot = (s == m).astype(jnp.float32)
        t4 = jax.lax.dot_general(
            k5, onehot, (((0,), (0,)), ((), ())),
            preferred_element_type=jnp.float32,
        )
        tgt = t4[0:3, :] / t4[3:4, :]
        toff = tgt - qt
        mag = jnp.sqrt(jnp.sum(toff * toff, axis=0, keepdims=True))
        x = predt_ref[:, cols] - toff
        ax = jnp.abs(x)
        e = jnp.where(ax < 1.0, 0.5 * x * x, ax - 0.5)
        elem = jnp.sum(e, axis=0, keepdims=True)
        r0 = (pid * qb + h * sb) // lanes
        mag_scr[pl.ds(r0, sb // lanes), :] = mag.reshape(sb // lanes, lanes)
        elem_scr[pl.ds(r0, sb // lanes), :] = elem.reshape(sb // lanes, lanes)

    @pl.when(pid == n_steps - 1)
    def _loss():
        mag = mag_scr[...]
        bits = jax.lax.bitcast_convert_type(mag, jnp.int32)

        def body(_, lohi):
            lo, hi = lohi
            mid = lo + (hi - lo) // 2
            cnt = jnp.sum((bits <= mid).astype(jnp.int32))
            take = cnt >= k_count
            return jnp.where(take, lo, mid + 1), jnp.where(take, mid, hi)

        _, hi = jax.lax.fori_loop(
            0, 31, body, (jnp.int32(0), jnp.int32(0x7F800000))
        )
        thresh = jax.lax.bitcast_convert_type(hi, jnp.float32)
        mask = (mag <= thresh).astype(jnp.float32)
        denom = jnp.maximum(jnp.sum(mask) * 3.0, 1.0)
        loss = jnp.sum(elem_scr[...] * mask) / denom
        out_ref[...] = jnp.broadcast_to(loss, (1, 1))


@jax.jit
def kernel(pred_off, queries, keys):
    Q, D = queries.shape
    C = keys.shape[0]
    QB = 4096
    N_SUB = 4
    LANES = 1024
    keys_t = keys.T
    queries_t = queries.T
    pred_t = pred_off.T

    k_count = int(0.99 * (Q - 1)) + 1

    out = pl.pallas_call(
        functools.partial(_fused, Q // QB, N_SUB, k_count),
        grid=(Q // QB,),
        in_specs=[
            pl.BlockSpec((C, D), lambda i: (0, 0)),
            pl.BlockSpec((D, C), lambda i: (0, 0)),
            pl.BlockSpec((D, QB), lambda i: (0, i)),
            pl.BlockSpec((D, QB), lambda i: (0, i)),
        ],
        out_specs=pl.BlockSpec((1, 1), lambda i: (0, 0)),
        out_shape=jax.ShapeDtypeStruct((1, 1), jnp.float32),
        scratch_shapes=[
            pltpu.VMEM((Q // LANES, LANES), jnp.float32),
            pltpu.VMEM((Q // LANES, LANES), jnp.float32),
        ],
    )(keys, keys_t, queries_t, pred_t)
    return out[0, 0]
